# baseline (speedup 1.0000x reference)
import numpy as np

import jax
import jax.numpy as jnp
from jax import lax
from jax.experimental import pallas as pl
from jax.experimental.pallas import tpu as pltpu
from jax.experimental.pallas import tpu_sc as plsc

SEQ_LEN = 200
VOCAB = 100000
DIM = 64
BATCH = 4096

NC = 2
NS = 16
LANES = 16
NW = NC * NS
BTILE = BATCH // NW
ST = SEQ_LEN // 8
DT = DIM // 8
GROUPS = DIM // LANES
N_PAIRS = SEQ_LEN // 2

PITCH = BTILE + 1
_TIDX = np.zeros((GROUPS + 1, LANES), np.int32)
_TIDX[:GROUPS] = (np.arange(GROUPS * LANES)
                  .reshape(GROUPS, LANES).astype(np.int32))


NBUF = 4
N_STEPS = SEQ_LEN // NBUF


def _body(idx_hbm, token_hbm, pos_hbm, tidx_hbm, out_hbm,
          idx_v, pos_v, tidx_v,
          rows_0, rows_1, rows_2, rows_3, out_0, out_1, out_2, out_3,
          gsem_0, gsem_1, gsem_2, gsem_3, osem_0, osem_1, osem_2, osem_3):
    c = lax.axis_index("c")
    s_ax = lax.axis_index("s")
    wid = s_ax * NC + c

    rows = [rows_0, rows_1, rows_2, rows_3]
    outs = [out_0, out_1, out_2, out_3]
    gsems = [gsem_0, gsem_1, gsem_2, gsem_3]
    osems = [osem_0, osem_1, osem_2, osem_3]

    pltpu.sync_copy(idx_hbm.at[:, wid, :, :], idx_v)
    pltpu.sync_copy(pos_hbm, pos_v)
    pltpu.sync_copy(tidx_hbm, tidx_v)

    def fire_gather(u, buf, sem):
        pltpu.async_copy(
            token_hbm.at[idx_v.at[u // 8, lax.rem(u, 8), :]], buf, sem)

    def drain(buf, sem):
        pltpu.make_async_copy(token_hbm.at[idx_v.at[0, 0, :]], buf, sem).wait()

    def drain_out(out_t, sem):
        for dt in range(DT):
            pltpu.make_async_copy(out_hbm.at[0, dt, 0],
                                  out_t.at[pl.ds(dt * 8, 8), pl.ds(0, BTILE)],
                                  sem).wait()

    def transpose_add(u, rows_t, out_t):
        pvs = [pos_v[u, pl.ds(g * LANES, LANES)] for g in range(GROUPS)]
        dvecs = [tidx_v[g, :] for g in range(GROUPS)]
        zero = tidx_v[GROUPS, :]

        @plsc.parallel_loop(0, BTILE, unroll=2)
        def _(bb):
            bbv = zero + bb
            for g in range(GROUPS):
                val = rows_t[bb, pl.ds(g * LANES, LANES)] + pvs[g]
                plsc.store_scatter(out_t, [dvecs[g], bbv], val)

    def writeout(u, out_t, sem):
        for dt in range(DT):
            pltpu.async_copy(out_t.at[pl.ds(dt * 8, 8), pl.ds(0, BTILE)],
                             out_hbm.at[u, dt, wid], sem)

    for k in range(NBUF):
        fire_gather(k, rows[k], gsems[k])

    def step(t, carry):
        for k in range(NBUF):
            u = NBUF * t + k
            drain(rows[k], gsems[k])

            @pl.when(t > 0)
            def _():
                drain_out(outs[k], osems[k])
            transpose_add(u, rows[k], outs[k])

            @pl.when(t < N_STEPS - 1)
            def _():
                fire_gather(u + NBUF, rows[k], gsems[k])
            writeout(u, outs[k], osems[k])
        return carry

    lax.fori_loop(0, N_STEPS, step, 0)
    for k in range(NBUF):
        drain_out(outs[k], osems[k])


@jax.jit
def _run(idx4, token_table, position_table, tidx):
    mesh = plsc.VectorSubcoreMesh(
        core_axis_name="c", subcore_axis_name="s",
        num_cores=NC, num_subcores=NS,
    )
    fn = pl.kernel(
        _body,
        out_type=jax.ShapeDtypeStruct((SEQ_LEN, DT, NW, 8, BTILE), jnp.float32),
        mesh=mesh,
        compiler_params=pltpu.CompilerParams(
            use_tc_tiling_on_sc=False, needs_layout_passes=False),
        scratch_types=[
            pltpu.VMEM((ST, 8, BTILE), jnp.int32),
            pltpu.VMEM((SEQ_LEN, DIM), jnp.float32),
            pltpu.VMEM((GROUPS + 1, LANES), jnp.int32),
        ] + [pltpu.VMEM((BTILE, DIM), jnp.float32)] * NBUF
          + [pltpu.VMEM((DIM, PITCH), jnp.float32)] * NBUF
          + [pltpu.SemaphoreType.DMA] * (2 * NBUF),
    )
    return fn(idx4, token_table, position_table, tidx)


def kernel(inputs, token_table, position_table):
    idx = inputs.astype(jnp.int32)
    idx4 = idx.T.reshape(ST, 8, NW, BTILE).transpose(0, 2, 1, 3)
    out5 = _run(idx4, token_table, position_table, jnp.asarray(_TIDX))
    return out5.transpose(2, 4, 0, 1, 3).reshape(BATCH, SEQ_LEN, DIM)


# speedup vs baseline: 1.0017x; 1.0017x over previous
import numpy as np

import jax
import jax.numpy as jnp
from jax import lax
from jax.experimental import pallas as pl
from jax.experimental.pallas import tpu as pltpu
from jax.experimental.pallas import tpu_sc as plsc

SEQ_LEN = 200
VOCAB = 100000
DIM = 64
BATCH = 4096

NC = 2
NS = 16
LANES = 16
NW = NC * NS
BTILE = BATCH // NW
ST = SEQ_LEN // 8
DT = DIM // 8
GROUPS = DIM // LANES
N_PAIRS = SEQ_LEN // 2

PITCH = BTILE + 1
_TIDX = np.zeros((GROUPS + 1, LANES), np.int32)
_TIDX[:GROUPS] = (np.arange(GROUPS * LANES)
                  .reshape(GROUPS, LANES).astype(np.int32))


NBUF = 4
N_STEPS = SEQ_LEN // NBUF


def _body(idx_hbm, token_hbm, pos_hbm, tidx_hbm, out_hbm,
          idx_v, pos_v, tidx_v,
          rows_0, rows_1, rows_2, rows_3, out_0, out_1, out_2, out_3,
          gsem_0, gsem_1, gsem_2, gsem_3, osem_0, osem_1, osem_2, osem_3):
    c = lax.axis_index("c")
    s_ax = lax.axis_index("s")
    wid = s_ax * NC + c

    rows = [rows_0, rows_1, rows_2, rows_3]
    outs = [out_0, out_1, out_2, out_3]
    gsems = [gsem_0, gsem_1, gsem_2, gsem_3]
    osems = [osem_0, osem_1, osem_2, osem_3]

    pltpu.sync_copy(idx_hbm.at[:, wid, :, :], idx_v)
    pltpu.sync_copy(pos_hbm, pos_v)
    pltpu.sync_copy(tidx_hbm, tidx_v)

    def fire_gather(u, buf, sem):
        pltpu.async_copy(
            token_hbm.at[idx_v.at[u // 8, lax.rem(u, 8), :]], buf, sem)

    def drain(buf, sem):
        pltpu.make_async_copy(token_hbm.at[idx_v.at[0, 0, :]], buf, sem).wait()

    def drain_out(out_t, sem):
        for dt in range(DT):
            pltpu.make_async_copy(out_hbm.at[0, dt, 0],
                                  out_t.at[pl.ds(dt * 8, 8), pl.ds(0, BTILE)],
                                  sem).wait()

    def transpose_add(u, rows_t, out_t):
        pvs = [pos_v[u, pl.ds(g * LANES, LANES)] for g in range(GROUPS)]
        dvecs = [tidx_v[g, :] for g in range(GROUPS)]
        zero = tidx_v[GROUPS, :]

        @plsc.parallel_loop(0, BTILE, unroll=4)
        def _(bb):
            bbv = zero + bb
            for g in range(GROUPS):
                val = rows_t[bb, pl.ds(g * LANES, LANES)] + pvs[g]
                plsc.store_scatter(out_t, [dvecs[g], bbv], val)

    def writeout(u, out_t, sem):
        for dt in range(DT):
            pltpu.async_copy(out_t.at[pl.ds(dt * 8, 8), pl.ds(0, BTILE)],
                             out_hbm.at[u, dt, wid], sem)

    for k in range(NBUF):
        fire_gather(k, rows[k], gsems[k])

    def step(t, carry):
        for k in range(NBUF):
            u = NBUF * t + k
            drain(rows[k], gsems[k])

            @pl.when(t > 0)
            def _():
                drain_out(outs[k], osems[k])
            transpose_add(u, rows[k], outs[k])

            @pl.when(t < N_STEPS - 1)
            def _():
                fire_gather(u + NBUF, rows[k], gsems[k])
            writeout(u, outs[k], osems[k])
        return carry

    lax.fori_loop(0, N_STEPS, step, 0)
    for k in range(NBUF):
        drain_out(outs[k], osems[k])


@jax.jit
def _run(idx4, token_table, position_table, tidx):
    mesh = plsc.VectorSubcoreMesh(
        core_axis_name="c", subcore_axis_name="s",
        num_cores=NC, num_subcores=NS,
    )
    fn = pl.kernel(
        _body,
        out_type=jax.ShapeDtypeStruct((SEQ_LEN, DT, NW, 8, BTILE), jnp.float32),
        mesh=mesh,
        compiler_params=pltpu.CompilerParams(
            use_tc_tiling_on_sc=False, needs_layout_passes=False),
        scratch_types=[
            pltpu.VMEM((ST, 8, BTILE), jnp.int32),
            pltpu.VMEM((SEQ_LEN, DIM), jnp.float32),
            pltpu.VMEM((GROUPS + 1, LANES), jnp.int32),
        ] + [pltpu.VMEM((BTILE, DIM), jnp.float32)] * NBUF
          + [pltpu.VMEM((DIM, PITCH), jnp.float32)] * NBUF
          + [pltpu.SemaphoreType.DMA] * (2 * NBUF),
    )
    return fn(idx4, token_table, position_table, tidx)


def kernel(inputs, token_table, position_table):
    idx = inputs.astype(jnp.int32)
    idx4 = idx.T.reshape(ST, 8, NW, BTILE).transpose(0, 2, 1, 3)
    out5 = _run(idx4, token_table, position_table, jnp.asarray(_TIDX))
    return out5.transpose(2, 4, 0, 1, 3).reshape(BATCH, SEQ_LEN, DIM)
